# baseline (speedup 1.0000x reference)
import functools

import jax
import jax.numpy as jnp
from jax import lax
from jax.experimental import pallas as pl
from jax.experimental.pallas import tpu as pltpu
from jax.experimental.pallas import tpu_sc as plsc

_T, _H, _W = 16, 32, 32
_D3 = 256
_D = 3 * _D3
_NROW = _T + _H + _W
_NPOS = 4 * 8192
_NC, _NS, _L = 2, 16, 16
_NW = _NC * _NS
_PER_W = _NPOS // _NW
_CHUNK = 32
_NCH = _PER_W // _CHUNK
_NBUF = 4
_SLOT = _CHUNK * _D


def _emb_body(tab, ids, out, tabv, ids_v, rowb, wsem):
    cid = lax.axis_index("c")
    sid = lax.axis_index("s")
    wid = sid * _NC + cid
    base = wid * _PER_W

    pltpu.sync_copy(tab, tabv)
    pltpu.sync_copy(ids.at[pl.ds(base, _PER_W)], ids_v)

    def chunk_body(c, _):
        boff = (c & (_NBUF - 1)) * _SLOT

        @pl.when(c >= _NBUF)
        def _wait_oldest():
            pltpu.make_async_copy(
                rowb.at[pl.ds(0, _SLOT)], out.at[pl.ds(0, _SLOT)],
                wsem).wait()

        for kk in range(_CHUNK // _L):
            pvec = ids_v[pl.ds(c * _CHUNK + kk * _L, _L)]
            r0v = (pvec >> 10) * _D3
            r1v = (((pvec >> 5) & (_H - 1)) + _T) * _D3
            r2v = ((pvec & (_W - 1)) + _T + _H) * _D3
            obase = boff + kk * _L * _D
            for l in range(_L):
                rows = (r0v[l], r1v[l], r2v[l])
                o = obase + l * _D
                for dim, roff in enumerate(rows):
                    for v in range(_D3 // _L):
                        rowb[pl.ds(o + dim * _D3 + v * _L, _L)] = (
                            tabv[pl.ds(roff + v * _L, _L)])

        pltpu.make_async_copy(
            rowb.at[pl.ds(boff, _SLOT)],
            out.at[pl.ds((base + c * _CHUNK) * _D, _SLOT)],
            wsem).start()
        return 0

    lax.fori_loop(0, _NCH, chunk_body, 0)

    for _ in range(_NBUF):
        pltpu.make_async_copy(
            rowb.at[pl.ds(0, _SLOT)], out.at[pl.ds(0, _SLOT)], wsem).wait()


@functools.partial(
    pl.kernel,
    mesh=plsc.VectorSubcoreMesh(core_axis_name="c", subcore_axis_name="s"),
    out_type=jax.ShapeDtypeStruct((_NPOS * _D,), jnp.float32),
    scratch_types=[
        pltpu.VMEM((_NROW * _D3,), jnp.float32),
        pltpu.VMEM((_PER_W,), jnp.int32),
        pltpu.VMEM((_NBUF * _SLOT,), jnp.float32),
        pltpu.SemaphoreType.DMA,
    ],
    compiler_params=pltpu.CompilerParams(needs_layout_passes=False),
)
def _emb_kernel(tab, ids, out, *scratch):
    _emb_body(tab, ids, out, *scratch)


def kernel(d_0, d_1, d_2, position_ids):
    B, Lseq = position_ids.shape
    ids = position_ids.reshape(-1).astype(jnp.int32)
    tab = jnp.concatenate([d_0, d_1, d_2], axis=0).reshape(-1)
    out = _emb_kernel(tab, ids)
    return out.reshape(B, Lseq, _D)


# speedup vs baseline: 1.1940x; 1.1940x over previous
import functools

import jax
import jax.numpy as jnp
from jax import lax
from jax.experimental import pallas as pl
from jax.experimental.pallas import tpu as pltpu
from jax.experimental.pallas import tpu_sc as plsc

_T, _H, _W = 16, 32, 32
_D3 = 256
_D = 3 * _D3
_NROW = _T + _H + _W
_NPOS = 4 * 8192
_NC, _NS, _L = 2, 16, 16
_NW = _NC * _NS
_PER_W = _NPOS // _NW
_CHUNK = 32
_NCH = _PER_W // _CHUNK
_NBUF = 4
_SLOT = _CHUNK * _D


def _emb_body(tab, ids, out, tabv, ids_v, rowb, wsem):
    cid = lax.axis_index("c")
    sid = lax.axis_index("s")
    wid = sid * _NC + cid
    base = wid * _PER_W

    pltpu.sync_copy(tab, tabv)
    pltpu.sync_copy(ids.at[pl.ds(base, _PER_W)], ids_v)

    def chunk_body(c, _):
        boff = (c & (_NBUF - 1)) * _SLOT

        @pl.when(c >= _NBUF)
        def _wait_oldest():
            pltpu.make_async_copy(
                rowb.at[pl.ds(0, _SLOT)], out.at[pl.ds(0, _SLOT)],
                wsem).wait()

        for kk in range(_CHUNK // _L):
            pvec = ids_v[pl.ds(c * _CHUNK + kk * _L, _L)]
            r0v = (pvec >> 10) * _D3
            r1v = (((pvec >> 5) & (_H - 1)) + _T) * _D3
            r2v = ((pvec & (_W - 1)) + _T + _H) * _D3
            obase = boff + kk * _L * _D
            for l in range(_L):
                rows = (r0v[l], r1v[l], r2v[l])
                o = obase + l * _D
                vals = [
                    tabv[pl.ds(roff + v * _L, _L)]
                    for roff in rows
                    for v in range(_D3 // _L)
                ]
                for i, val in enumerate(vals):
                    rowb[pl.ds(o + i * _L, _L)] = val

        pltpu.make_async_copy(
            rowb.at[pl.ds(boff, _SLOT)],
            out.at[pl.ds((base + c * _CHUNK) * _D, _SLOT)],
            wsem).start()
        return 0

    lax.fori_loop(0, _NCH, chunk_body, 0)

    for _ in range(_NBUF):
        pltpu.make_async_copy(
            rowb.at[pl.ds(0, _SLOT)], out.at[pl.ds(0, _SLOT)], wsem).wait()


@functools.partial(
    pl.kernel,
    mesh=plsc.VectorSubcoreMesh(core_axis_name="c", subcore_axis_name="s"),
    out_type=jax.ShapeDtypeStruct((_NPOS * _D,), jnp.float32),
    scratch_types=[
        pltpu.VMEM((_NROW * _D3,), jnp.float32),
        pltpu.VMEM((_PER_W,), jnp.int32),
        pltpu.VMEM((_NBUF * _SLOT,), jnp.float32),
        pltpu.SemaphoreType.DMA,
    ],
    compiler_params=pltpu.CompilerParams(needs_layout_passes=False),
)
def _emb_kernel(tab, ids, out, *scratch):
    _emb_body(tab, ids, out, *scratch)


def kernel(d_0, d_1, d_2, position_ids):
    B, Lseq = position_ids.shape
    ids = position_ids.reshape(-1).astype(jnp.int32)
    tab = jnp.concatenate([d_0, d_1, d_2], axis=0).reshape(-1)
    out = _emb_kernel(tab, ids)
    return out.reshape(B, Lseq, _D)


# speedup vs baseline: 2.1364x; 1.7893x over previous
import functools

import jax
import jax.numpy as jnp
from jax import lax
from jax.experimental import pallas as pl
from jax.experimental.pallas import tpu as pltpu
from jax.experimental.pallas import tpu_sc as plsc

_T, _H, _W = 16, 32, 32
_D3 = 256
_D = 3 * _D3
_NROW = _T + _H + _W
_NPOS = 4 * 8192
_NC, _NS, _L = 2, 16, 16
_NW = _NC * _NS
_PER_W = _NPOS // _NW


def _emb_body(tab, ids, out, tabv, ids_v, dummyv, wsem):
    cid = lax.axis_index("c")
    sid = lax.axis_index("s")
    wid = sid * _NC + cid
    base = wid * _PER_W

    pltpu.sync_copy(tab, tabv)
    pltpu.sync_copy(ids.at[pl.ds(base, _PER_W)], ids_v)

    def group_body(g, _):
        @pl.when(g >= 1)
        def _drain_prev():
            pltpu.make_async_copy(
                out.at[pl.ds(0, _L * _D)], dummyv, wsem).wait()

        pvec = ids_v[pl.ds(g * _L, _L)]
        r0v = pvec >> 10
        r1v = ((pvec >> 5) & (_H - 1)) + _T
        r2v = (pvec & (_W - 1)) + _T + _H
        obase = (base + g * _L) * _D
        for l in range(_L):
            o = pl.multiple_of(obase + l * _D, _D3)
            pltpu.make_async_copy(
                tabv.at[r0v[l]], out.at[pl.ds(o, _D3)], wsem).start()
            pltpu.make_async_copy(
                tabv.at[r1v[l]], out.at[pl.ds(o + _D3, _D3)], wsem).start()
            pltpu.make_async_copy(
                tabv.at[r2v[l]], out.at[pl.ds(o + 2 * _D3, _D3)], wsem).start()
        return 0

    lax.fori_loop(0, _PER_W // _L, group_body, 0)

    pltpu.make_async_copy(
        out.at[pl.ds(0, _L * _D)], dummyv, wsem).wait()


@functools.partial(
    pl.kernel,
    mesh=plsc.VectorSubcoreMesh(core_axis_name="c", subcore_axis_name="s"),
    out_type=jax.ShapeDtypeStruct((_NPOS * _D,), jnp.float32),
    scratch_types=[
        pltpu.VMEM((_NROW, _D3), jnp.float32),
        pltpu.VMEM((_PER_W,), jnp.int32),
        pltpu.VMEM((_L * _D,), jnp.float32),
        pltpu.SemaphoreType.DMA,
    ],
    compiler_params=pltpu.CompilerParams(needs_layout_passes=False),
)
def _emb_kernel(tab, ids, out, *scratch):
    _emb_body(tab, ids, out, *scratch)


def kernel(d_0, d_1, d_2, position_ids):
    B, Lseq = position_ids.shape
    ids = position_ids.reshape(-1).astype(jnp.int32)
    tab = jnp.concatenate([d_0, d_1, d_2], axis=0)
    out = _emb_kernel(tab, ids)
    return out.reshape(B, Lseq, _D)
